# baseline (speedup 1.0000x reference)
import functools

import jax
import jax.numpy as jnp
from jax import lax
from jax.experimental import pallas as pl
from jax.experimental.pallas import tpu as pltpu
from jax.experimental.pallas import tpu_sc as plsc

_LANES = 16
_POS_CHUNK = 32
_D = 1024

_CHUNK_ELEMS = _POS_CHUNK * _D
_GROUPS = _CHUNK_ELEMS // _LANES


def _sc_body(x_hbm, pos_hbm, out_hbm, xbuf, tbuf):
    wid = lax.axis_index("s") * 2 + lax.axis_index("c")

    batch = x_hbm.shape[0]
    seq_elems = x_hbm.shape[1]
    per_worker = seq_elems // 32
    chunks = per_worker // _CHUNK_ELEMS
    base = wid * per_worker

    def add_group(g, _):
        sl = pl.ds(g * _LANES, _LANES)
        plsc.addupdate(xbuf.at[sl], tbuf[sl])
        return _

    def chunk_loop(c, _):
        off = base + c * _CHUNK_ELEMS
        pltpu.sync_copy(pos_hbm.at[pl.ds(off, _CHUNK_ELEMS)], tbuf)

        def batch_loop(b, _):
            pltpu.sync_copy(x_hbm.at[b, pl.ds(off, _CHUNK_ELEMS)], xbuf)
            lax.fori_loop(0, _GROUPS, add_group, None, unroll=8)
            pltpu.sync_copy(xbuf, out_hbm.at[b, pl.ds(off, _CHUNK_ELEMS)])
            return _

        lax.fori_loop(0, batch, batch_loop, None)
        return _

    lax.fori_loop(0, chunks, chunk_loop, None)


def kernel(x, pos_table):
    batch, seq_len, d_model = x.shape
    x2 = x.reshape(batch, seq_len * d_model)
    pos2 = pos_table[:seq_len].reshape(seq_len * d_model)

    mesh = plsc.VectorSubcoreMesh(core_axis_name="c", subcore_axis_name="s")
    run = functools.partial(
        pl.kernel,
        mesh=mesh,
        out_type=jax.ShapeDtypeStruct((batch, seq_len * d_model), x.dtype),
        scratch_types=[
            pltpu.VMEM((_CHUNK_ELEMS,), jnp.float32),
            pltpu.VMEM((_CHUNK_ELEMS,), jnp.float32),
        ],
    )(_sc_body)
    out = run(x2, pos2)
    return out.reshape(batch, seq_len, d_model)


# speedup vs baseline: 1.0464x; 1.0464x over previous
import functools

import jax
import jax.numpy as jnp
from jax import lax
from jax.experimental import pallas as pl
from jax.experimental.pallas import tpu as pltpu
from jax.experimental.pallas import tpu_sc as plsc

_LANES = 16
_POS_CHUNK = 32
_WORKERS = 32


def _sc_body(x_hbm, pos_hbm, out_hbm, xbuf, tbuf):
    wid = lax.axis_index("s") * 2 + lax.axis_index("c")

    batch, seq_len, d_model = x_hbm.shape
    per_worker = seq_len // _WORKERS
    chunks = per_worker // _POS_CHUNK
    base = wid * per_worker
    groups_per_row = d_model // _LANES

    def add_row(r, _):
        def add_group(g, _):
            sl = pl.ds(g * _LANES, _LANES)
            plsc.addupdate(xbuf.at[r, sl], tbuf[r, sl])
            return _

        lax.fori_loop(0, groups_per_row, add_group, None, unroll=8)
        return _

    def chunk_loop(c, _):
        s0 = base + c * _POS_CHUNK
        pltpu.sync_copy(pos_hbm.at[pl.ds(s0, _POS_CHUNK)], tbuf)

        def batch_loop(b, _):
            pltpu.sync_copy(x_hbm.at[b, pl.ds(s0, _POS_CHUNK)], xbuf)
            lax.fori_loop(0, _POS_CHUNK, add_row, None)
            pltpu.sync_copy(xbuf, out_hbm.at[b, pl.ds(s0, _POS_CHUNK)])
            return _

        lax.fori_loop(0, batch, batch_loop, None)
        return _

    lax.fori_loop(0, chunks, chunk_loop, None)


def kernel(x, pos_table):
    batch, seq_len, d_model = x.shape

    mesh = plsc.VectorSubcoreMesh(core_axis_name="c", subcore_axis_name="s")
    run = functools.partial(
        pl.kernel,
        mesh=mesh,
        out_type=jax.ShapeDtypeStruct((batch, seq_len, d_model), x.dtype),
        scratch_types=[
            pltpu.VMEM((_POS_CHUNK, d_model), jnp.float32),
            pltpu.VMEM((_POS_CHUNK, d_model), jnp.float32),
        ],
    )(_sc_body)
    return run(x, pos_table[:seq_len])


# speedup vs baseline: 1.2633x; 1.2072x over previous
import functools

import jax
import jax.numpy as jnp
from jax import lax
from jax.experimental import pallas as pl
from jax.experimental.pallas import tpu as pltpu
from jax.experimental.pallas import tpu_sc as plsc

_LANES = 16
_CHUNK = 16
_WORKERS = 32
_NBUF = 3


def _sc_body(x_hbm, pos_hbm, out_hbm,
             t0, t1, x0, x1, x2,
             st0, st1, si0, si1, si2, so0, so1, so2):
    wid = lax.axis_index("s") * 2 + lax.axis_index("c")

    batch, seq_len, d_model = x_hbm.shape
    per_worker = seq_len // _WORKERS
    nchunks = per_worker // _CHUNK
    nitems = nchunks * batch
    base = wid * per_worker
    groups = d_model // _LANES

    tbufs, tsems = [t0, t1], [st0, st1]
    xbufs, isems, osems = [x0, x1, x2], [si0, si1, si2], [so0, so1, so2]

    in_h = [None] * nitems
    out_h = [None] * nitems
    t_h = [None] * nchunks

    def start_in(i):
        c, b = divmod(i, batch)
        s0 = base + c * _CHUNK
        in_h[i] = pltpu.async_copy(
            x_hbm.at[b, pl.ds(s0, _CHUNK)], xbufs[i % _NBUF], isems[i % _NBUF])

    def start_t(c):
        s0 = base + c * _CHUNK
        t_h[c] = pltpu.async_copy(
            pos_hbm.at[pl.ds(s0, _CHUNK)], tbufs[c % 2], tsems[c % 2])

    def start_out(i):
        c, b = divmod(i, batch)
        s0 = base + c * _CHUNK
        out_h[i] = pltpu.async_copy(
            xbufs[i % _NBUF], out_hbm.at[b, pl.ds(s0, _CHUNK)], osems[i % _NBUF])

    def add_item(i, c):
        xb, tb = xbufs[i % _NBUF], tbufs[c % 2]

        def add_row(r, carry):
            def add_group(g, carry):
                sl = pl.ds(g * _LANES, _LANES)
                plsc.addupdate(xb.at[r, sl], tb[r, sl])
                return carry

            lax.fori_loop(0, groups, add_group, None, unroll=8)
            return carry

        lax.fori_loop(0, _CHUNK, add_row, None)

    start_t(0)
    start_t(1)
    start_in(0)
    start_in(1)

    for i in range(nitems):
        c, b = divmod(i, batch)
        if b == 0 and 1 <= c and c + 1 < nchunks:
            start_t(c + 1)
        if b == 0:
            t_h[c].wait()
        j = i + 2
        if j < nitems:
            if j - _NBUF >= 0:
                out_h[j - _NBUF].wait()
            start_in(j)
        in_h[i].wait()
        add_item(i, c)
        start_out(i)

    for i in range(nitems - _NBUF, nitems):
        out_h[i].wait()


def kernel(x, pos_table):
    batch, seq_len, d_model = x.shape

    mesh = plsc.VectorSubcoreMesh(core_axis_name="c", subcore_axis_name="s")
    run = functools.partial(
        pl.kernel,
        mesh=mesh,
        out_type=jax.ShapeDtypeStruct((batch, seq_len, d_model), x.dtype),
        scratch_types=[
            pltpu.VMEM((_CHUNK, d_model), jnp.float32),
            pltpu.VMEM((_CHUNK, d_model), jnp.float32),
            pltpu.VMEM((_CHUNK, d_model), jnp.float32),
            pltpu.VMEM((_CHUNK, d_model), jnp.float32),
            pltpu.VMEM((_CHUNK, d_model), jnp.float32),
            pltpu.SemaphoreType.DMA,
            pltpu.SemaphoreType.DMA,
            pltpu.SemaphoreType.DMA,
            pltpu.SemaphoreType.DMA,
            pltpu.SemaphoreType.DMA,
            pltpu.SemaphoreType.DMA,
            pltpu.SemaphoreType.DMA,
            pltpu.SemaphoreType.DMA,
        ],
    )(_sc_body)
    return run(x, pos_table[:seq_len])
